# baseline (speedup 1.0000x reference)
import functools
import jax
import jax.numpy as jnp
from jax import lax
from jax.experimental import pallas as pl
from jax.experimental.pallas import tpu as pltpu, tpu_sc as plsc

EMBED_DIM = 512
BATCH = 1024
_NUM_CORES = 2
_NUM_SUBCORES = 16
_NUM_WORKERS = _NUM_CORES * _NUM_SUBCORES
_B_PER_W = BATCH // _NUM_WORKERS
_HALF = _B_PER_W // 2

_mesh = plsc.VectorSubcoreMesh(core_axis_name="c", subcore_axis_name="s")

@functools.partial(
    pl.kernel,
    mesh=_mesh,
    out_type=jax.ShapeDtypeStruct((BATCH, EMBED_DIM), jnp.float32),
    scratch_types=[
        pltpu.VMEM((_B_PER_W,), jnp.int32),
        pltpu.VMEM((_HALF, EMBED_DIM), jnp.float32),
        pltpu.VMEM((_HALF, EMBED_DIM), jnp.float32),
        pltpu.SemaphoreType.DMA,
        pltpu.SemaphoreType.DMA,
        pltpu.SemaphoreType.DMA,
        pltpu.SemaphoreType.DMA,
    ],
)
def _gather_rows(table_hbm, idx_hbm, out_hbm, idx_v, rows0, rows1, g0, g1, s0, s1):
    wid = lax.axis_index("s") * _NUM_CORES + lax.axis_index("c")
    base = wid * _B_PER_W
    pltpu.sync_copy(idx_hbm.at[pl.ds(base, _B_PER_W)], idx_v)
    c0 = pltpu.async_copy(table_hbm.at[idx_v.at[pl.ds(0, _HALF)]], rows0, g0)
    c1 = pltpu.async_copy(table_hbm.at[idx_v.at[pl.ds(_HALF, _HALF)]], rows1, g1)
    c0.wait()
    w0 = pltpu.async_copy(rows0, out_hbm.at[pl.ds(base, _HALF)], s0)
    c1.wait()
    w1 = pltpu.async_copy(rows1, out_hbm.at[pl.ds(base + _HALF, _HALF)], s1)
    w0.wait()
    w1.wait()

def kernel(x, t, embeddings):
    return _gather_rows(embeddings, t.astype(jnp.int32))
